# baseline (speedup 1.0000x reference)
import functools

import jax
import jax.numpy as jnp
from jax import lax
from jax.experimental import pallas as pl
from jax.experimental.pallas import tpu as pltpu
from jax.experimental.pallas import tpu_sc as plsc

N = 256
B = 1024
K = N * (N + 1) // 2
NWORKERS = 32
BPW = B // NWORKERS
L = 16
R = 64
NJ = N // R
RT = R // 8
W = [N - R * j for j in range(NJ)]
OFF = [R * j * N - (R * j) * (R * j - 1) // 2 for j in range(NJ)]
OB = K + R
assert all(w % L == 0 for w in W)


def _spd_body(x_hbm, out_hbm, inbuf, obuf0, obuf1, sem_in0, sem_in1,
              sem_out0, sem_out1):
    wid = lax.axis_index("s") * 2 + lax.axis_index("c")
    b0 = wid * BPW
    sems_in = (sem_in0, sem_in1)
    sems_out = (sem_out0, sem_out1)
    obufs = (obuf0, obuf1)
    iota = lax.iota(jnp.int32, L)

    def issue_in(b, j):
        c0 = 128 if R * j >= 128 else 0
        src = x_hbm.at[b0 + b, pl.ds(RT * j, RT), :, pl.ds(c0, N - c0)]
        dst = inbuf.at[j % 2, :, :, pl.ds(c0, N - c0)]
        pltpu.async_copy(src, dst, sems_in[j % 2])

    def wait_in(j):
        c0 = 128 if R * j >= 128 else 0
        src = x_hbm.at[0, pl.ds(RT * j, RT), :, pl.ds(c0, N - c0)]
        dst = inbuf.at[j % 2, :, :, pl.ds(c0, N - c0)]
        pltpu.make_async_copy(src, dst, sems_in[j % 2]).wait()

    def out_desc(u, off):
        return pltpu.make_async_copy(
            obufs[u].at[pl.ds(0, K)], out_hbm.at[pl.ds(off, K)], sems_out[u])

    def compute_chunk(u, j):
        w = W[j]
        ref3 = inbuf.at[j % 2]
        obuf = obufs[u]

        g = 2 if w > 128 else 4

        def row_group_body(p, ooff):
            li = g * p
            vs = []
            for s in range(g):
                lis = li + s
                rt = jnp.broadcast_to(lis // 8, (L,)).astype(jnp.int32)
                r = jnp.broadcast_to(lis % 8, (L,)).astype(jnp.int32)
                cvec = R * j + lis + iota
                vs.append([plsc.load_gather(ref3, [rt, r, cvec + L * t])
                           for t in range(w // L)])
            os_ = ooff
            for s in range(g):
                for t, v in enumerate(vs[s]):
                    obuf[pl.ds(os_ + L * t, L)] = v
                os_ = os_ + (w - li - s)
            return os_

        lax.fori_loop(0, R // g, row_group_body, OFF[j])

    issue_in(0, 0)

    def pair_body(bp, carry):
        for u in (0, 1):
            b = bp * 2 + u

            @pl.when(bp >= 1)
            def _():
                out_desc(u, 0).wait()

            for j in range(NJ):
                wait_in(j)
                if j < NJ - 1:
                    issue_in(b, j + 1)
                else:
                    issue_in(jnp.minimum(b + 1, BPW - 1), 0)
                compute_chunk(u, j)
            off = pl.multiple_of((b0 + b) * K, 8)
            out_desc(u, off).start()
        return carry

    lax.fori_loop(0, BPW // 2, pair_body, 0)
    out_desc(0, 0).wait()
    out_desc(1, 0).wait()
    wait_in(0)


def kernel(input):
    x = input.reshape(B, N // 8, 8, N)
    mesh = plsc.VectorSubcoreMesh(core_axis_name="c", subcore_axis_name="s")
    spd = functools.partial(
        pl.kernel,
        mesh=mesh,
        out_type=jax.ShapeDtypeStruct((B * K,), jnp.float32),
        compiler_params=pltpu.CompilerParams(needs_layout_passes=False),
        scratch_types=[
            pltpu.VMEM((2, RT, 8, N), jnp.float32),
            pltpu.VMEM((OB,), jnp.float32),
            pltpu.VMEM((OB,), jnp.float32),
            pltpu.SemaphoreType.DMA,
            pltpu.SemaphoreType.DMA,
            pltpu.SemaphoreType.DMA,
            pltpu.SemaphoreType.DMA,
        ],
    )(_spd_body)
    return spd(x).reshape(B, K)


# speedup vs baseline: 1.0010x; 1.0010x over previous
import functools

import jax
import jax.numpy as jnp
from jax import lax
from jax.experimental import pallas as pl
from jax.experimental.pallas import tpu as pltpu
from jax.experimental.pallas import tpu_sc as plsc

N = 256
B = 1024
K = N * (N + 1) // 2
NWORKERS = 32
BPW = B // NWORKERS
L = 16
R = 64
NJ = N // R
RT = R // 8
W = [N - R * j for j in range(NJ)]
OFF = [R * j * N - (R * j) * (R * j - 1) // 2 for j in range(NJ)]
OB = K + R
assert all(w % L == 0 for w in W)


def _spd_body(x_hbm, out_hbm, inbuf, obuf0, obuf1, sem_in0, sem_in1,
              sem_out0, sem_out1):
    wid = lax.axis_index("s") * 2 + lax.axis_index("c")
    b0 = wid * BPW
    sems_in = (sem_in0, sem_in1)
    sems_out = (sem_out0, sem_out1)
    obufs = (obuf0, obuf1)
    iota = lax.iota(jnp.int32, L)

    def issue_in(b, j):
        c0 = 128 if R * j >= 128 else 0
        src = x_hbm.at[b0 + b, pl.ds(RT * j, RT), :, pl.ds(c0, N - c0)]
        dst = inbuf.at[j % 2, :, :, pl.ds(c0, N - c0)]
        pltpu.async_copy(src, dst, sems_in[j % 2])

    def wait_in(j):
        c0 = 128 if R * j >= 128 else 0
        src = x_hbm.at[0, pl.ds(RT * j, RT), :, pl.ds(c0, N - c0)]
        dst = inbuf.at[j % 2, :, :, pl.ds(c0, N - c0)]
        pltpu.make_async_copy(src, dst, sems_in[j % 2]).wait()

    def out_desc(u, off):
        return pltpu.make_async_copy(
            obufs[u].at[pl.ds(0, K)], out_hbm.at[pl.ds(off, K)], sems_out[u])

    def compute_chunk(u, j):
        w = W[j]
        ref3 = inbuf.at[j % 2]
        obuf = obufs[u]

        g = 2

        def row_group_body(p, ooff):
            li = g * p
            vs = []
            for s in range(g):
                lis = li + s
                rt = jnp.broadcast_to(lis // 8, (L,)).astype(jnp.int32)
                r = jnp.broadcast_to(lis % 8, (L,)).astype(jnp.int32)
                cvec = R * j + lis + iota
                vs.append([plsc.load_gather(ref3, [rt, r, cvec + L * t])
                           for t in range(w // L)])
            os_ = ooff
            for s in range(g):
                for t, v in enumerate(vs[s]):
                    obuf[pl.ds(os_ + L * t, L)] = v
                os_ = os_ + (w - li - s)
            return os_

        lax.fori_loop(0, R // g, row_group_body, OFF[j])

    issue_in(0, 0)

    def pair_body(bp, carry):
        for u in (0, 1):
            b = bp * 2 + u

            @pl.when(bp >= 1)
            def _():
                out_desc(u, 0).wait()

            for j in range(NJ):
                wait_in(j)
                if j < NJ - 1:
                    issue_in(b, j + 1)
                else:
                    issue_in(jnp.minimum(b + 1, BPW - 1), 0)
                compute_chunk(u, j)
            off = pl.multiple_of((b0 + b) * K, 8)
            out_desc(u, off).start()
        return carry

    lax.fori_loop(0, BPW // 2, pair_body, 0)
    out_desc(0, 0).wait()
    out_desc(1, 0).wait()
    wait_in(0)


def kernel(input):
    x = input.reshape(B, N // 8, 8, N)
    mesh = plsc.VectorSubcoreMesh(core_axis_name="c", subcore_axis_name="s")
    spd = functools.partial(
        pl.kernel,
        mesh=mesh,
        out_type=jax.ShapeDtypeStruct((B * K,), jnp.float32),
        compiler_params=pltpu.CompilerParams(needs_layout_passes=False),
        scratch_types=[
            pltpu.VMEM((2, RT, 8, N), jnp.float32),
            pltpu.VMEM((OB,), jnp.float32),
            pltpu.VMEM((OB,), jnp.float32),
            pltpu.SemaphoreType.DMA,
            pltpu.SemaphoreType.DMA,
            pltpu.SemaphoreType.DMA,
            pltpu.SemaphoreType.DMA,
        ],
    )(_spd_body)
    return spd(x).reshape(B, K)
